# baseline (speedup 1.0000x reference)
import functools

import jax
import jax.numpy as jnp
from jax import lax
from jax.experimental import pallas as pl
from jax.experimental.pallas import tpu as pltpu
from jax.experimental.pallas import tpu_sc as plsc

_NC = 2
_NS = 16
_NW = _NC * _NS
_L = 16

_CH = 80


def _gae_decode(z, src_idx, dst_idx):
    n, d = z.shape
    e = src_idx.shape[0]
    epw = e // _NW
    nchunk = epw // _CH
    groups = _CH // _L

    mesh = plsc.VectorSubcoreMesh(core_axis_name="c", subcore_axis_name="s")

    @functools.partial(
        pl.kernel,
        mesh=mesh,
        compiler_params=pltpu.CompilerParams(needs_layout_passes=False),
        out_type=jax.ShapeDtypeStruct((e,), jnp.float32),
        scratch_types=[
            pltpu.VMEM((_CH,), jnp.int32),
            pltpu.VMEM((_CH,), jnp.int32),
            pltpu.VMEM((_CH, d), jnp.float32),
            pltpu.VMEM((_CH, d), jnp.float32),
            pltpu.VMEM((_CH,), jnp.float32),
            pltpu.SemaphoreType.DMA,
            pltpu.SemaphoreType.DMA,
        ],
    )
    def decode(z_hbm, sidx_hbm, didx_hbm, out_hbm,
               sidx_v, didx_v, srows_v, drows_v, outc_v, sem_s, sem_d):
        wid = lax.axis_index("s") * _NC + lax.axis_index("c")
        wbase = wid * epw

        def chunk_body(c, carry):
            base = wbase + c * _CH
            pltpu.sync_copy(sidx_hbm.at[pl.ds(base, _CH)], sidx_v)
            pltpu.sync_copy(didx_hbm.at[pl.ds(base, _CH)], didx_v)
            cp_s = pltpu.async_copy(z_hbm.at[sidx_v], srows_v, sem_s)
            cp_d = pltpu.async_copy(z_hbm.at[didx_v], drows_v, sem_d)
            cp_s.wait()
            cp_d.wait()

            def group_body(g, carry):
                e_vec = lax.iota(jnp.int32, _L) + g * _L
                accs = [jnp.zeros((_L,), jnp.float32) for _ in range(4)]
                for k in range(d):
                    d_vec = jnp.full((_L,), k, jnp.int32)
                    sv = plsc.load_gather(srows_v, [e_vec, d_vec])
                    dv = plsc.load_gather(drows_v, [e_vec, d_vec])
                    accs[k % 4] = accs[k % 4] + sv * dv
                acc = (accs[0] + accs[1]) + (accs[2] + accs[3])
                outc_v[pl.ds(g * _L, _L)] = 1.0 / (1.0 + jnp.exp(-acc))
                return carry

            lax.fori_loop(0, groups, group_body, 0)
            pltpu.sync_copy(outc_v, out_hbm.at[pl.ds(base, _CH)])
            return carry

        lax.fori_loop(0, nchunk, chunk_body, 0)

    return decode(z, src_idx, dst_idx)


def kernel(z, edge_index):
    ei = edge_index.astype(jnp.int32)
    return _gae_decode(z.astype(jnp.float32), ei[0], ei[1])


# speedup vs baseline: 2.2766x; 2.2766x over previous
import functools

import jax
import jax.numpy as jnp
from jax import lax
from jax.experimental import pallas as pl
from jax.experimental.pallas import tpu as pltpu
from jax.experimental.pallas import tpu_sc as plsc

_NC = 2
_NS = 16
_NW = _NC * _NS
_L = 16

_CH = 80


def _gae_decode(z, src_idx, dst_idx):
    n, d = z.shape
    e = src_idx.shape[0]
    epw = e // _NW
    nchunk = epw // _CH
    groups = _CH // _L

    mesh = plsc.VectorSubcoreMesh(core_axis_name="c", subcore_axis_name="s")

    @functools.partial(
        pl.kernel,
        mesh=mesh,
        compiler_params=pltpu.CompilerParams(needs_layout_passes=False),
        out_type=jax.ShapeDtypeStruct((e,), jnp.float32),
        scratch_types=[
            pltpu.VMEM((_CH,), jnp.int32),
            pltpu.VMEM((_CH,), jnp.int32),
            pltpu.VMEM((_CH, d), jnp.float32),
            pltpu.VMEM((_CH, d), jnp.float32),
            pltpu.VMEM((_CH,), jnp.float32),
            pltpu.SemaphoreType.DMA,
            pltpu.SemaphoreType.DMA,
        ],
    )
    def decode(z_hbm, sidx_hbm, didx_hbm, out_hbm,
               sidx_v, didx_v, srows_v, drows_v, outc_v, sem_s, sem_d):
        wid = lax.axis_index("s") * _NC + lax.axis_index("c")
        wbase = wid * epw

        def chunk_body(c, carry):
            base = wbase + c * _CH
            pltpu.sync_copy(sidx_hbm.at[pl.ds(base, _CH)], sidx_v)
            pltpu.sync_copy(didx_hbm.at[pl.ds(base, _CH)], didx_v)
            cp_s = pltpu.async_copy(z_hbm.at[sidx_v], srows_v, sem_s)
            cp_d = pltpu.async_copy(z_hbm.at[didx_v], drows_v, sem_d)
            cp_s.wait()
            cp_d.wait()

            def group_body(g, carry):
                e_vec = lax.iota(jnp.int32, _L) + g * _L
                accs = [jnp.zeros((_L,), jnp.float32) for _ in range(4)]
                lane = lax.iota(jnp.int32, _L)
                for k in range(d):
                    d_vec = jnp.bitwise_and(lane + k, d - 1)
                    sv = plsc.load_gather(srows_v, [e_vec, d_vec])
                    dv = plsc.load_gather(drows_v, [e_vec, d_vec])
                    accs[k % 4] = accs[k % 4] + sv * dv
                acc = (accs[0] + accs[1]) + (accs[2] + accs[3])
                outc_v[pl.ds(g * _L, _L)] = 1.0 / (1.0 + jnp.exp(-acc))
                return carry

            lax.fori_loop(0, groups, group_body, 0)
            pltpu.sync_copy(outc_v, out_hbm.at[pl.ds(base, _CH)])
            return carry

        lax.fori_loop(0, nchunk, chunk_body, 0)

    return decode(z, src_idx, dst_idx)


def kernel(z, edge_index):
    ei = edge_index.astype(jnp.int32)
    return _gae_decode(z.astype(jnp.float32), ei[0], ei[1])


# speedup vs baseline: 2.7763x; 1.2195x over previous
import functools

import jax
import jax.numpy as jnp
from jax import lax
from jax.experimental import pallas as pl
from jax.experimental.pallas import tpu as pltpu
from jax.experimental.pallas import tpu_sc as plsc

_NC = 2
_NS = 16
_NW = _NC * _NS
_L = 16

_CH = 80


def _gae_decode(z, src_idx, dst_idx):
    n, d = z.shape
    e = src_idx.shape[0]
    epw = e // _NW
    nchunk = epw // _CH
    groups = _CH // _L

    mesh = plsc.VectorSubcoreMesh(core_axis_name="c", subcore_axis_name="s")

    @functools.partial(
        pl.kernel,
        mesh=mesh,
        compiler_params=pltpu.CompilerParams(needs_layout_passes=False),
        out_type=jax.ShapeDtypeStruct((e,), jnp.float32),
        scratch_types=[
            pltpu.VMEM((_CH,), jnp.int32),
            pltpu.VMEM((_CH,), jnp.int32),
            pltpu.VMEM((_CH, d), jnp.float32),
            pltpu.VMEM((_CH, d), jnp.float32),
            pltpu.VMEM((_CH,), jnp.float32),
            pltpu.SemaphoreType.DMA,
            pltpu.SemaphoreType.DMA,
        ],
    )
    def decode(z_hbm, sidx_hbm, didx_hbm, out_hbm,
               sidx_v, didx_v, srows_v, drows_v, outc_v, sem_s, sem_d):
        wid = lax.axis_index("s") * _NC + lax.axis_index("c")
        wbase = wid * epw

        def chunk_body(c, carry):
            base = wbase + c * _CH
            pltpu.sync_copy(sidx_hbm.at[pl.ds(base, _CH)], sidx_v)
            pltpu.sync_copy(didx_hbm.at[pl.ds(base, _CH)], didx_v)
            cp_s = pltpu.async_copy(z_hbm.at[sidx_v], srows_v, sem_s)
            cp_d = pltpu.async_copy(z_hbm.at[didx_v], drows_v, sem_d)
            cp_s.wait()
            cp_d.wait()

            lane = lax.iota(jnp.int32, _L)

            def group_body(g, carry):
                e0 = g * _L
                res = jnp.zeros((_L,), jnp.float32)
                for j in range(_L):
                    e = e0 + j
                    prods = []
                    for k in range(d // _L):
                        sl = pl.ds(k * _L, _L)
                        prods.append(srows_v[e, sl] * drows_v[e, sl])
                    while len(prods) > 1:
                        prods = [a + b for a, b in
                                 zip(prods[::2], prods[1::2])]
                    tot = jnp.sum(prods[0])
                    res = jnp.where(lane == j, tot, res)
                outc_v[pl.ds(e0, _L)] = 1.0 / (1.0 + jnp.exp(-res))
                return carry

            lax.fori_loop(0, groups, group_body, 0)
            pltpu.sync_copy(outc_v, out_hbm.at[pl.ds(base, _CH)])
            return carry

        lax.fori_loop(0, nchunk, chunk_body, 0)

    return decode(z, src_idx, dst_idx)


def kernel(z, edge_index):
    ei = edge_index.astype(jnp.int32)
    return _gae_decode(z.astype(jnp.float32), ei[0], ei[1])


# speedup vs baseline: 4.5382x; 1.6346x over previous
import functools

import jax
import jax.numpy as jnp
from jax import lax
from jax.experimental import pallas as pl
from jax.experimental.pallas import tpu as pltpu
from jax.experimental.pallas import tpu_sc as plsc

_NC = 2
_NS = 16
_NW = _NC * _NS
_L = 16

_CH = 80


def _gae_decode(z, src_idx, dst_idx):
    n, d = z.shape
    e = src_idx.shape[0]
    epw = e // _NW
    nchunk = epw // _CH
    groups = _CH // _L

    mesh = plsc.VectorSubcoreMesh(core_axis_name="c", subcore_axis_name="s")

    @functools.partial(
        pl.kernel,
        mesh=mesh,
        compiler_params=pltpu.CompilerParams(needs_layout_passes=False),
        out_type=jax.ShapeDtypeStruct((e,), jnp.float32),
        scratch_types=[
            pltpu.VMEM((epw,), jnp.int32),
            pltpu.VMEM((epw,), jnp.int32),
            pltpu.VMEM((_CH, d), jnp.float32),
            pltpu.VMEM((_CH, d), jnp.float32),
            pltpu.VMEM((_CH, d), jnp.float32),
            pltpu.VMEM((_CH, d), jnp.float32),
            pltpu.VMEM((epw,), jnp.float32),
            pltpu.SemaphoreType.DMA,
            pltpu.SemaphoreType.DMA,
            pltpu.SemaphoreType.DMA,
            pltpu.SemaphoreType.DMA,
        ],
    )
    def decode(z_hbm, sidx_hbm, didx_hbm, out_hbm,
               sidx_v, didx_v, srows0, srows1, drows0, drows1, out_v,
               sem_s0, sem_s1, sem_d0, sem_d1):
        wid = lax.axis_index("s") * _NC + lax.axis_index("c")
        wbase = wid * epw

        pltpu.sync_copy(sidx_hbm.at[pl.ds(wbase, epw)], sidx_v)
        pltpu.sync_copy(didx_hbm.at[pl.ds(wbase, epw)], didx_v)

        sbufs = (srows0, srows1)
        dbufs = (drows0, drows1)
        ssems = (sem_s0, sem_s1)
        dsems = (sem_d0, sem_d1)
        lane = lax.iota(jnp.int32, _L)

        def issue(c, b):
            isl = pl.ds(c * _CH, _CH)
            pltpu.async_copy(z_hbm.at[sidx_v.at[isl]], sbufs[b], ssems[b])
            pltpu.async_copy(z_hbm.at[didx_v.at[isl]], dbufs[b], dsems[b])

        def wait(b):
            pltpu.make_async_copy(
                z_hbm.at[sidx_v.at[pl.ds(0, _CH)]], sbufs[b], ssems[b]
            ).wait()
            pltpu.make_async_copy(
                z_hbm.at[didx_v.at[pl.ds(0, _CH)]], dbufs[b], dsems[b]
            ).wait()

        def compute(c, b):
            srows_v = sbufs[b]
            drows_v = dbufs[b]

            def group_body(g, carry):
                e0 = g * _L
                res = jnp.zeros((_L,), jnp.float32)
                for j in range(_L):
                    ee = e0 + j
                    prods = []
                    for k in range(d // _L):
                        sl = pl.ds(k * _L, _L)
                        prods.append(srows_v[ee, sl] * drows_v[ee, sl])
                    while len(prods) > 1:
                        prods = [a + b2 for a, b2 in
                                 zip(prods[::2], prods[1::2])]
                    tot = jnp.sum(prods[0])
                    res = jnp.where(lane == j, tot, res)
                out_v[pl.ds(c * _CH + e0, _L)] = 1.0 / (1.0 + jnp.exp(-res))
                return carry

            lax.fori_loop(0, groups, group_body, 0)

        issue(0, 0)
        issue(1, 1)

        def pair_body(i, carry):
            a = i * 2
            wait(0)
            compute(a, 0)

            @pl.when(a + 2 < nchunk)
            def _():
                issue(a + 2, 0)

            wait(1)
            compute(a + 1, 1)

            @pl.when(a + 3 < nchunk)
            def _():
                issue(a + 3, 1)

            return carry

        lax.fori_loop(0, nchunk // 2, pair_body, 0)
        if nchunk % 2:
            wait(0)
            compute(nchunk - 1, 0)

        pltpu.sync_copy(out_v, out_hbm.at[pl.ds(wbase, epw)])

    return decode(z, src_idx, dst_idx)


def kernel(z, edge_index):
    ei = edge_index.astype(jnp.int32)
    return _gae_decode(z.astype(jnp.float32), ei[0], ei[1])


# speedup vs baseline: 9.5301x; 2.1000x over previous
import functools

import jax
import jax.numpy as jnp
from jax import lax
from jax.experimental import pallas as pl
from jax.experimental.pallas import tpu as pltpu
from jax.experimental.pallas import tpu_sc as plsc

_NC = 2
_NS = 16
_NW = _NC * _NS
_L = 16

_CH = 80
_NBUF = 4


def _sq_norms(z):
    n, d = z.shape

    def body(z_ref, o_ref):
        zz = z_ref[...]
        o_ref[...] = jnp.sum(zz * zz, axis=1)

    return pl.pallas_call(
        body, out_shape=jax.ShapeDtypeStruct((n,), jnp.float32)
    )(z)


def _gae_decode(z, n2, src_idx, dst_idx):
    n, d = z.shape
    e = src_idx.shape[0]
    epw = e // _NW
    nchunk = epw // _CH
    groups = _CH // _L

    mesh = plsc.VectorSubcoreMesh(core_axis_name="c", subcore_axis_name="s")

    @functools.partial(
        pl.kernel,
        mesh=mesh,
        compiler_params=pltpu.CompilerParams(needs_layout_passes=False),
        out_type=jax.ShapeDtypeStruct((e,), jnp.float32),
        scratch_types=[
            pltpu.VMEM((epw,), jnp.int32),
            pltpu.VMEM((epw,), jnp.int32),
            pltpu.VMEM((n,), jnp.float32),
            pltpu.VMEM((epw,), jnp.float32),
        ]
        + [pltpu.VMEM((_CH, d), jnp.float32) for _ in range(_NBUF)]
        + [pltpu.SemaphoreType.DMA for _ in range(_NBUF)],
    )
    def decode(z_hbm, n2_hbm, sidx_hbm, didx_hbm, out_hbm,
               sidx_v, didx_v, n2_v, out_v,
               buf0, buf1, buf2, buf3, sem0, sem1, sem2, sem3):
        wid = lax.axis_index("s") * _NC + lax.axis_index("c")
        wbase = wid * epw

        pltpu.sync_copy(sidx_hbm.at[pl.ds(wbase, epw)], sidx_v)
        pltpu.sync_copy(didx_hbm.at[pl.ds(wbase, epw)], didx_v)
        pltpu.sync_copy(n2_hbm, n2_v)

        bufs = (buf0, buf1, buf2, buf3)
        sems = (sem0, sem1, sem2, sem3)
        lane = lax.iota(jnp.int32, _L)

        def issue_g1(c, b):
            pltpu.async_copy(z_hbm.at[sidx_v.at[pl.ds(c * _CH, _CH)]],
                             bufs[b], sems[b])

        def issue_add(c, b):
            pltpu.async_copy(z_hbm.at[didx_v.at[pl.ds(c * _CH, _CH)]],
                             bufs[b], sems[b], add=True)

        def wait(b):
            pltpu.make_async_copy(
                z_hbm.at[sidx_v.at[pl.ds(0, _CH)]], bufs[b], sems[b]
            ).wait()

        def compute(c, b):
            rows = bufs[b]

            def group_body(g, carry):
                e0 = g * _L
                esl = pl.ds(c * _CH + e0, _L)
                sivec = sidx_v[esl]
                divec = didx_v[esl]
                n2s = plsc.load_gather(n2_v, [sivec])
                n2d = plsc.load_gather(n2_v, [divec])
                res = jnp.zeros((_L,), jnp.float32)
                for j in range(_L):
                    ee = e0 + j
                    prods = []
                    for k in range(d // _L):
                        sl = pl.ds(k * _L, _L)
                        v = rows[ee, sl]
                        prods.append(v * v)
                    while len(prods) > 1:
                        prods = [a + b2 for a, b2 in
                                 zip(prods[::2], prods[1::2])]
                    tot = jnp.sum(prods[0])
                    res = jnp.where(lane == j, tot, res)
                dot = (res - n2s - n2d) * 0.5
                out_v[esl] = 1.0 / (1.0 + jnp.exp(-dot))
                return carry

            lax.fori_loop(0, groups, group_body, 0)

        issue_g1(0, 0)
        issue_g1(1, 1)
        wait(0)
        issue_add(0, 0)
        wait(1)
        issue_add(1, 1)
        issue_g1(2, 2)
        issue_g1(3, 3)

        def step(x, b):
            @pl.when(x + 2 < nchunk)
            def _():
                wait((b + 2) % _NBUF)
                issue_add(x + 2, (b + 2) % _NBUF)

            wait(b)
            compute(x, b)

            @pl.when(x + 4 < nchunk)
            def _():
                issue_g1(x + 4, b)

        def quad_body(i, carry):
            for t in range(_NBUF):
                step(i * _NBUF + t, t)
            return carry

        lax.fori_loop(0, nchunk // _NBUF, quad_body, 0)
        for t in range(nchunk % _NBUF):
            step((nchunk // _NBUF) * _NBUF + t, t)

        pltpu.sync_copy(out_v, out_hbm.at[pl.ds(wbase, epw)])

    return decode(z, n2, src_idx, dst_idx)


def kernel(z, edge_index):
    zf = z.astype(jnp.float32)
    ei = edge_index.astype(jnp.int32)
    return _gae_decode(zf, _sq_norms(zf), ei[0], ei[1])


# speedup vs baseline: 10.0591x; 1.0555x over previous
import functools

import jax
import jax.numpy as jnp
from jax import lax
from jax.experimental import pallas as pl
from jax.experimental.pallas import tpu as pltpu
from jax.experimental.pallas import tpu_sc as plsc

_NC = 2
_NS = 16
_NW = _NC * _NS
_L = 16

_CH = 80
_NBUF = 8
_H = _NBUF // 2


def _sq_norms(z):
    n, d = z.shape

    def body(z_ref, o_ref):
        zz = z_ref[...]
        o_ref[...] = jnp.sum(zz * zz, axis=1)

    return pl.pallas_call(
        body, out_shape=jax.ShapeDtypeStruct((n,), jnp.float32)
    )(z)


def _gae_decode(z, n2, src_idx, dst_idx):
    n, d = z.shape
    e = src_idx.shape[0]
    epw = e // _NW
    nchunk = epw // _CH
    groups = _CH // _L

    mesh = plsc.VectorSubcoreMesh(core_axis_name="c", subcore_axis_name="s")

    @functools.partial(
        pl.kernel,
        mesh=mesh,
        compiler_params=pltpu.CompilerParams(needs_layout_passes=False),
        out_type=jax.ShapeDtypeStruct((e,), jnp.float32),
        scratch_types=[
            pltpu.VMEM((epw,), jnp.int32),
            pltpu.VMEM((epw,), jnp.int32),
            pltpu.VMEM((n,), jnp.float32),
            pltpu.VMEM((epw,), jnp.float32),
        ]
        + [pltpu.VMEM((_CH, d), jnp.float32) for _ in range(_NBUF)]
        + [pltpu.SemaphoreType.DMA for _ in range(_NBUF)],
    )
    def decode(z_hbm, n2_hbm, sidx_hbm, didx_hbm, out_hbm,
               sidx_v, didx_v, n2_v, out_v, *bufs_and_sems):
        wid = lax.axis_index("s") * _NC + lax.axis_index("c")
        wbase = wid * epw

        pltpu.sync_copy(sidx_hbm.at[pl.ds(wbase, epw)], sidx_v)
        pltpu.sync_copy(didx_hbm.at[pl.ds(wbase, epw)], didx_v)
        pltpu.sync_copy(n2_hbm, n2_v)

        bufs = bufs_and_sems[:_NBUF]
        sems = bufs_and_sems[_NBUF:]
        lane = lax.iota(jnp.int32, _L)

        def issue_g1(c, b):
            pltpu.async_copy(z_hbm.at[sidx_v.at[pl.ds(c * _CH, _CH)]],
                             bufs[b], sems[b])

        def issue_add(c, b):
            pltpu.async_copy(z_hbm.at[didx_v.at[pl.ds(c * _CH, _CH)]],
                             bufs[b], sems[b], add=True)

        def wait(b):
            pltpu.make_async_copy(
                z_hbm.at[sidx_v.at[pl.ds(0, _CH)]], bufs[b], sems[b]
            ).wait()

        def compute(c, b):
            rows = bufs[b]

            def group_body(g, carry):
                e0 = g * _L
                esl = pl.ds(c * _CH + e0, _L)
                sivec = sidx_v[esl]
                divec = didx_v[esl]
                n2s = plsc.load_gather(n2_v, [sivec])
                n2d = plsc.load_gather(n2_v, [divec])
                res = jnp.zeros((_L,), jnp.float32)
                for j in range(_L):
                    ee = e0 + j
                    prods = []
                    for k in range(d // _L):
                        sl = pl.ds(k * _L, _L)
                        v = rows[ee, sl]
                        prods.append(v * v)
                    while len(prods) > 1:
                        prods = [a + b2 for a, b2 in
                                 zip(prods[::2], prods[1::2])]
                    tot = jnp.sum(prods[0])
                    res = jnp.where(lane == j, tot, res)
                dot = (res - n2s - n2d) * 0.5
                out_v[esl] = 1.0 / (1.0 + jnp.exp(-dot))
                return carry

            lax.fori_loop(0, groups, group_body, 0)

        for t in range(_H):
            issue_g1(t, t)
        for t in range(_H):
            wait(t)
            issue_add(t, t)
        for t in range(_H, _NBUF):
            issue_g1(t, t)

        def step(x, b):
            @pl.when(x + _H < nchunk)
            def _():
                wait((b + _H) % _NBUF)
                issue_add(x + _H, (b + _H) % _NBUF)

            wait(b)
            compute(x, b)

            @pl.when(x + _NBUF < nchunk)
            def _():
                issue_g1(x + _NBUF, b)

        def quad_body(i, carry):
            for t in range(_NBUF):
                step(i * _NBUF + t, t)
            return carry

        lax.fori_loop(0, nchunk // _NBUF, quad_body, 0)
        for t in range(nchunk % _NBUF):
            step((nchunk // _NBUF) * _NBUF + t, t)

        pltpu.sync_copy(out_v, out_hbm.at[pl.ds(wbase, epw)])

    return decode(z, n2, src_idx, dst_idx)


def kernel(z, edge_index):
    zf = z.astype(jnp.float32)
    ei = edge_index.astype(jnp.int32)
    return _gae_decode(zf, _sq_norms(zf), ei[0], ei[1])
